# Initial kernel scaffold; baseline (speedup 1.0000x reference)
import jax
import jax.numpy as jnp
from jax.experimental import pallas as pl


def kernel(h, edge_index):
    raise NotImplementedError("write your pallas kernel here")


# speedup vs baseline: 3.1958x; 3.1958x over previous
import functools

import jax
import jax.numpy as jnp
from jax import lax
from jax.experimental import pallas as pl
from jax.experimental.pallas import tpu as pltpu
from jax.experimental.pallas import tpu_sc as plsc

N_NODES = 10000
D = 128
E = 320000
LANES = 16
N_WORKERS = 32
E_PER_W = E // N_WORKERS
CHUNK = 80
N_CHUNKS = E_PER_W // CHUNK


def _sc_body(h_hbm, src_hbm, dst_hbm, out_hbm,
             src_v, dst_v, rs_v, rd_v, out_v, sem):
    wid = lax.axis_index("s") * 2 + lax.axis_index("c")
    base = wid * E_PER_W
    pltpu.sync_copy(src_hbm.at[pl.ds(base, E_PER_W)], src_v)
    pltpu.sync_copy(dst_hbm.at[pl.ds(base, E_PER_W)], dst_v)

    def chunk_body(j, carry):
        off = j * CHUNK
        cp1 = pltpu.async_copy(h_hbm.at[src_v.at[pl.ds(off, CHUNK)]], rs_v, sem)
        cp2 = pltpu.async_copy(h_hbm.at[dst_v.at[pl.ds(off, CHUNK)]], rd_v, sem)
        cp1.wait()
        cp2.wait()

        lane0 = lax.iota(jnp.int32, LANES) == 0

        def edge_body(e, c):
            acc = rs_v[e, pl.ds(0, LANES)] * rd_v[e, pl.ds(0, LANES)]
            for k in range(1, D // LANES):
                acc = acc + (rs_v[e, pl.ds(k * LANES, LANES)] *
                             rd_v[e, pl.ds(k * LANES, LANES)])
            s = jnp.maximum(jnp.sum(acc), -0.9)
            idx = lax.broadcast(off + e, (LANES,))
            plsc.store_scatter(out_v, [idx], lax.broadcast(s, (LANES,)),
                               mask=lane0)
            return c

        lax.fori_loop(0, CHUNK, edge_body, 0, unroll=False)
        return carry

    lax.fori_loop(0, N_CHUNKS, chunk_body, 0, unroll=False)
    pltpu.sync_copy(out_v, out_hbm.at[pl.ds(base, E_PER_W)])


@functools.partial(
    pl.kernel,
    mesh=plsc.VectorSubcoreMesh(core_axis_name="c", subcore_axis_name="s"),
    compiler_params=pltpu.CompilerParams(needs_layout_passes=False),
    out_type=jax.ShapeDtypeStruct((E,), jnp.float32),
    scratch_types=[
        pltpu.VMEM((E_PER_W,), jnp.int32),
        pltpu.VMEM((E_PER_W,), jnp.int32),
        pltpu.VMEM((CHUNK, D), jnp.float32),
        pltpu.VMEM((CHUNK, D), jnp.float32),
        pltpu.VMEM((E_PER_W,), jnp.float32),
        pltpu.SemaphoreType.DMA,
    ],
)
def _sc_kernel(h_hbm, src_hbm, dst_hbm, out_hbm,
               src_v, dst_v, rs_v, rd_v, out_v, sem):
    _sc_body(h_hbm, src_hbm, dst_hbm, out_hbm,
             src_v, dst_v, rs_v, rd_v, out_v, sem)


def kernel(h, edge_index):
    ei = edge_index.astype(jnp.int32)
    out = _sc_kernel(h, ei[0], ei[1])
    return out.reshape(E, 1)


# speedup vs baseline: 6.5670x; 2.0549x over previous
import functools

import jax
import jax.numpy as jnp
from jax import lax
from jax.experimental import pallas as pl
from jax.experimental.pallas import tpu as pltpu
from jax.experimental.pallas import tpu_sc as plsc

N_NODES = 10000
D = 128
E = 320000
LANES = 16
N_WORKERS = 32
E_PER_W = E // N_WORKERS
CHUNK = 80
N_CHUNKS = E_PER_W // CHUNK


def _sc_body(h_hbm, src_hbm, dst_hbm, out_hbm,
             src_v, dst_v, rs_v, rd_v, out_v, sems):
    wid = lax.axis_index("s") * 2 + lax.axis_index("c")
    base = wid * E_PER_W
    pltpu.sync_copy(src_hbm.at[pl.ds(base, E_PER_W)], src_v)
    pltpu.sync_copy(dst_hbm.at[pl.ds(base, E_PER_W)], dst_v)

    def start(j, par):
        off = j * CHUNK
        pltpu.async_copy(h_hbm.at[src_v.at[pl.ds(off, CHUNK)]],
                         rs_v.at[par], sems.at[par])
        pltpu.async_copy(h_hbm.at[dst_v.at[pl.ds(off, CHUNK)]],
                         rd_v.at[par], sems.at[par])

    start(0, 0)
    lane15 = lax.iota(jnp.int32, LANES) == (LANES - 1)

    def chunk_body(j, carry):
        par = lax.rem(j, 2)
        off = j * CHUNK

        @pl.when(j + 1 < N_CHUNKS)
        def _():
            start(j + 1, 1 - par)

        pltpu.make_async_copy(h_hbm.at[pl.ds(0, CHUNK)],
                              rs_v.at[par], sems.at[par]).wait()
        pltpu.make_async_copy(h_hbm.at[pl.ds(0, CHUNK)],
                              rd_v.at[par], sems.at[par]).wait()

        def edge_body(e, c):
            acc = rs_v[par, e, pl.ds(0, LANES)] * rd_v[par, e, pl.ds(0, LANES)]
            for k in range(1, D // LANES):
                acc = acc + (rs_v[par, e, pl.ds(k * LANES, LANES)] *
                             rd_v[par, e, pl.ds(k * LANES, LANES)])
            s = jnp.maximum(plsc.cumsum(acc), -0.9)
            idx = lax.broadcast(off + e, (LANES,))
            plsc.store_scatter(out_v, [idx], s, mask=lane15)
            return c

        lax.fori_loop(0, CHUNK, edge_body, 0, unroll=False)
        return carry

    lax.fori_loop(0, N_CHUNKS, chunk_body, 0, unroll=False)
    pltpu.sync_copy(out_v, out_hbm.at[pl.ds(base, E_PER_W)])


@functools.partial(
    pl.kernel,
    mesh=plsc.VectorSubcoreMesh(core_axis_name="c", subcore_axis_name="s"),
    compiler_params=pltpu.CompilerParams(needs_layout_passes=False),
    out_type=jax.ShapeDtypeStruct((E,), jnp.float32),
    scratch_types=[
        pltpu.VMEM((E_PER_W,), jnp.int32),
        pltpu.VMEM((E_PER_W,), jnp.int32),
        pltpu.VMEM((2, CHUNK, D), jnp.float32),
        pltpu.VMEM((2, CHUNK, D), jnp.float32),
        pltpu.VMEM((E_PER_W,), jnp.float32),
        pltpu.SemaphoreType.DMA((2,)),
    ],
)
def _sc_kernel(h_hbm, src_hbm, dst_hbm, out_hbm,
               src_v, dst_v, rs_v, rd_v, out_v, sems):
    _sc_body(h_hbm, src_hbm, dst_hbm, out_hbm,
             src_v, dst_v, rs_v, rd_v, out_v, sems)


def kernel(h, edge_index):
    ei = edge_index.astype(jnp.int32)
    out = _sc_kernel(h, ei[0], ei[1])
    return out.reshape(E, 1)
